# baseline (speedup 1.0000x reference)
import functools
import math

import jax
import jax.numpy as jnp
from jax import lax
from jax.experimental import pallas as pl
from jax.experimental.pallas import tpu as pltpu
from jax.experimental.pallas import tpu_sc as plsc

N0 = 10000
E = 320000
EPAD = 327680
ECH = 80
EROWS = EPAD // 128
NW = 32

P0 = 10112
K1 = 8000
P1 = 8064
K2 = 5600
P2 = 5632
KP = 8192

_f32 = jnp.float32
_i32 = jnp.int32


def _gelu(v):
    return 0.5 * v * (1.0 + lax.erf(v * (1.0 / math.sqrt(2.0))))


def _h0_body(x_ref, w_ref, b_ref, wrel_ref, h_ref, g_ref):
    i = pl.program_id(0)
    h = jnp.dot(x_ref[...], w_ref[...], preferred_element_type=_f32, precision=lax.Precision.HIGHEST) + b_ref[...]
    h = _gelu(h)
    rv = i * 128 + lax.broadcasted_iota(_i32, (128, 1), 0)
    h = jnp.where(rv < N0, h, 0.0)
    h_ref[...] = h
    g_ref[...] = jnp.dot(h, wrel_ref[...], preferred_element_type=_f32, precision=lax.Precision.HIGHEST)


def _h0_call(x_pad, wd, bd, wrel):
    return pl.pallas_call(
        _h0_body,
        grid=(P0 // 128,),
        in_specs=[
            pl.BlockSpec((128, 128), lambda i: (i, 0)),
            pl.BlockSpec((128, 16), lambda i: (0, 0)),
            pl.BlockSpec((1, 16), lambda i: (0, 0)),
            pl.BlockSpec((16, 128), lambda i: (0, 0)),
        ],
        out_specs=[
            pl.BlockSpec((128, 16), lambda i: (i, 0)),
            pl.BlockSpec((128, 128), lambda i: (i, 0)),
        ],
        out_shape=[
            jax.ShapeDtypeStruct((P0, 16), _f32),
            jax.ShapeDtypeStruct((P0, 128), _f32),
        ],
    )(x_pad, wd, bd, wrel)


def _make_sc_agg(P, F):
    rp = P // 16
    mesh = plsc.VectorSubcoreMesh(core_axis_name="c", subcore_axis_name="s",
                                  num_cores=2, num_subcores=16)

    def body(tab, src2, dst2, zb, out, acc_sh,
             idxs, idxd, rows0, rows1, sem0, sem1):
        c = lax.axis_index("c")
        s = lax.axis_index("s")
        w = s * 2 + c
        pltpu.sync_copy(zb, acc_sh.at[pl.ds(s * rp, rp)])
        plsc.subcore_barrier()

        def batch(bt, carry):
            pltpu.sync_copy(src2.at[pl.ds(w * ECH + bt * 16, 16)], idxs)
            pltpu.sync_copy(dst2.at[pl.ds(w * ECH + bt * 16, 16)], idxd)

            def step(j, c2):
                pltpu.async_copy(tab.at[idxs.at[2 * j]], rows0, sem0)
                pltpu.async_copy(tab.at[idxs.at[2 * j + 1]], rows1, sem1)
                pltpu.make_async_copy(tab.at[idxs.at[2 * j]], rows0,
                                      sem0).wait()
                pltpu.sync_copy(rows0, acc_sh.at[idxd.at[2 * j]], add=True)
                pltpu.make_async_copy(tab.at[idxs.at[2 * j + 1]], rows1,
                                      sem1).wait()
                pltpu.sync_copy(rows1, acc_sh.at[idxd.at[2 * j + 1]], add=True)
                return c2

            lax.fori_loop(0, 8, step, 0)
            return carry

        lax.fori_loop(0, ECH // 16, batch, 0)
        plsc.subcore_barrier()
        pltpu.sync_copy(acc_sh.at[pl.ds(s * rp, rp)],
                        out.at[c, pl.ds(s * rp, rp)])

    return pl.kernel(
        body,
        out_type=jax.ShapeDtypeStruct((2, P, F), _f32),
        mesh=mesh,
        scratch_types=[
            pltpu.VMEM_SHARED((P, F), _f32),
            pltpu.VMEM((16, 128), _i32),
            pltpu.VMEM((16, 128), _i32),
            pltpu.VMEM((128, F), _f32),
            pltpu.VMEM((128, F), _f32),
            pltpu.SemaphoreType.DMA,
            pltpu.SemaphoreType.DMA,
        ],
        compiler_params=pltpu.CompilerParams(needs_layout_passes=False),
    )


def _make_conv(P, Fin, Fout, nvalid, pre_projected=False):
    def body(agg_ref, t_ref, wr_ref, wo_ref, b_ref, pw_ref, h_ref, s_ref):
        i = pl.program_id(0)
        a = agg_ref[0] + agg_ref[1]
        if pre_projected:
            rel = a
        else:
            rel = jnp.dot(a, wr_ref[...], preferred_element_type=_f32, precision=lax.Precision.HIGHEST)
        h = (rel + b_ref[...]
             + jnp.dot(t_ref[...], wo_ref[...], preferred_element_type=_f32, precision=lax.Precision.HIGHEST))
        h = _gelu(h)
        h_ref[...] = h
        w = pw_ref[...]
        norm = jnp.sqrt(jnp.sum(w * w))
        sc = jnp.tanh(jnp.sum(h * w, axis=1, keepdims=True) / norm)
        rv = i * 128 + lax.broadcasted_iota(_i32, (128, 1), 0)
        s_ref[...] = jnp.where(rv < nvalid, sc, -2.0)

    Fa = Fout if pre_projected else Fin

    def call(agg, tab, wr, wo, b, pw):
        return pl.pallas_call(
            body,
            grid=(P // 128,),
            in_specs=[
                pl.BlockSpec((2, 128, Fa), lambda i: (0, i, 0)),
                pl.BlockSpec((128, Fin), lambda i: (i, 0)),
                pl.BlockSpec((Fin, Fout), lambda i: (0, 0)),
                pl.BlockSpec((Fin, Fout), lambda i: (0, 0)),
                pl.BlockSpec((1, Fout), lambda i: (0, 0)),
                pl.BlockSpec((1, Fout), lambda i: (0, 0)),
            ],
            out_specs=[
                pl.BlockSpec((128, Fout), lambda i: (i, 0)),
                pl.BlockSpec((128, 1), lambda i: (i, 0)),
            ],
            out_shape=[
                jax.ShapeDtypeStruct((P, Fout), _f32),
                jax.ShapeDtypeStruct((P, 1), _f32),
            ],
        )(agg, tab, wr, wo, b, pw)

    return call


def _make_topk(R, K):
    N = R * 128
    m = int(math.log2(N))
    assert 1 << m == N

    def body(s_ref, so_ref, io_ref):
        row = lax.broadcasted_iota(_i32, (R, 128), 0)
        col = lax.broadcasted_iota(_i32, (R, 128), 1)
        idx = row * 128 + col
        cs = s_ref[...]
        ci = idx

        for ke in range(1, m + 1):
            for je in range(ke - 1, -1, -1):
                d = 1 << je
                lower = (idx & d) == 0
                up = (idx & (1 << ke)) == 0
                take_first = up == lower
                if d < 128:
                    bs_dn = pltpu.roll(cs, 128 - d, 1)
                    bs_up = pltpu.roll(cs, d, 1)
                    bi_dn = pltpu.roll(ci, 128 - d, 1)
                    bi_up = pltpu.roll(ci, d, 1)
                else:
                    dd = d // 128
                    bs_dn = jnp.concatenate([cs[dd:], cs[:dd]], axis=0)
                    bs_up = jnp.concatenate([cs[R - dd:], cs[:R - dd]], axis=0)
                    bi_dn = jnp.concatenate([ci[dd:], ci[:dd]], axis=0)
                    bi_up = jnp.concatenate([ci[R - dd:], ci[:R - dd]], axis=0)
                bs = jnp.where(lower, bs_dn, bs_up)
                bi = jnp.where(lower, bi_dn, bi_up)
                before = (cs > bs) | ((cs == bs) & (ci < bi))
                sel_a = before == take_first
                cs = jnp.where(sel_a, cs, bs)
                ci = jnp.where(sel_a, ci, bi)

        keep = idx < K
        so_ref[...] = jnp.where(keep, cs, 0.0)
        io_ref[...] = jnp.where(keep, ci, 0)

    def call(scores2d):
        return pl.pallas_call(
            body,
            out_shape=[
                jax.ShapeDtypeStruct((R, 128), _f32),
                jax.ShapeDtypeStruct((R, 128), _i32),
            ],
        )(scores2d)

    return call


def _make_sc_pool(NI, K, Pprev, F):
    mesh = plsc.VectorSubcoreMesh(core_axis_name="c", subcore_axis_name="s",
                                  num_cores=2, num_subcores=16)
    n16 = NI // 16
    k16 = K // 16
    g16 = ECH * 8

    def body(htab, perm2, src2, dst2, hg, srco, dsto,
             nidx_v, perm_v, ebs, ebd, eos, eod, rows, sem):
        c = lax.axis_index("c")
        s = lax.axis_index("s")
        w = s * 2 + c

        pltpu.sync_copy(perm2, perm_v)

        def fill(t, carry):
            nidx_v[pl.ds(t * 16, 16)] = jnp.full((16,), K, _i32)
            return carry

        lax.fori_loop(0, n16, fill, 0)

        def scat(t, carry):
            r = t // 8
            cseg = (t % 8) * 16
            pv = perm_v[r, pl.ds(cseg, 16)]
            vals = t * 16 + lax.iota(_i32, 16)
            plsc.store_scatter(nidx_v, [pv], vals)
            return carry

        lax.fori_loop(0, k16, scat, 0)

        pltpu.sync_copy(src2.at[pl.ds(w * ECH, ECH)], ebs)
        pltpu.sync_copy(dst2.at[pl.ds(w * ECH, ECH)], ebd)

        def remap(t, carry):
            r = t // 8
            cseg = (t % 8) * 16
            sv = ebs[r, pl.ds(cseg, 16)]
            eos[r, pl.ds(cseg, 16)] = plsc.load_gather(nidx_v, [sv])
            dv = ebd[r, pl.ds(cseg, 16)]
            eod[r, pl.ds(cseg, 16)] = plsc.load_gather(nidx_v, [dv])
            return carry

        lax.fori_loop(0, g16, remap, 0)

        pltpu.sync_copy(eos, srco.at[pl.ds(w * ECH, ECH)])
        pltpu.sync_copy(eod, dsto.at[pl.ds(w * ECH, ECH)])

        for t in range(KP // 128 // NW):
            cid = w * (KP // 128 // NW) + t
            pltpu.async_copy(htab.at[perm_v.at[cid]], rows, sem).wait()
            pltpu.sync_copy(rows, hg.at[pl.ds(cid * 128, 128)])

    return pl.kernel(
        body,
        out_type=[
            jax.ShapeDtypeStruct((KP, F), _f32),
            jax.ShapeDtypeStruct((EROWS, 128), _i32),
            jax.ShapeDtypeStruct((EROWS, 128), _i32),
        ],
        mesh=mesh,
        scratch_types=[
            pltpu.VMEM((NI,), _i32),
            pltpu.VMEM((KP // 128, 128), _i32),
            pltpu.VMEM((ECH, 128), _i32),
            pltpu.VMEM((ECH, 128), _i32),
            pltpu.VMEM((ECH, 128), _i32),
            pltpu.VMEM((ECH, 128), _i32),
            pltpu.VMEM((128, F), _f32),
            pltpu.SemaphoreType.DMA,
        ],
        compiler_params=pltpu.CompilerParams(needs_layout_passes=False),
    )


def _make_scale(P, F):
    def body(hg_ref, ts_ref, o_ref):
        o_ref[...] = hg_ref[...] * ts_ref[...]

    def call(hg, ts):
        return pl.pallas_call(
            body,
            grid=(P // 128,),
            in_specs=[
                pl.BlockSpec((128, F), lambda i: (i, 0)),
                pl.BlockSpec((128, 1), lambda i: (i, 0)),
            ],
            out_specs=pl.BlockSpec((128, F), lambda i: (i, 0)),
            out_shape=jax.ShapeDtypeStruct((P, F), _f32),
        )(hg, ts)

    return call


def _final_body(agga_ref, aggb_ref, t_ref, wr_ref, wo_ref, b_ref,
                wo1_ref, bo1_ref, wo2_ref, bo2_ref, o_ref, acc_ref):
    i = pl.program_id(0)
    nb = pl.num_programs(0)
    a = jnp.concatenate([agga_ref[0] + agga_ref[1],
                         aggb_ref[0] + aggb_ref[1]], axis=1)
    h = (jnp.dot(a, wr_ref[...], preferred_element_type=_f32, precision=lax.Precision.HIGHEST)
         + b_ref[...]
         + jnp.dot(t_ref[...], wo_ref[...], preferred_element_type=_f32, precision=lax.Precision.HIGHEST))
    h = _gelu(h)

    @pl.when(i == 0)
    def _init():
        acc_ref[...] = jnp.zeros((2, 512), _f32)

    rv = i * 128 + lax.broadcasted_iota(_i32, (128, 1), 0)
    valid = rv < K2
    evenm = jnp.where(valid & (rv % 2 == 0), 1.0, 0.0)
    oddm = jnp.where(valid & (rv % 2 == 1), 1.0, 0.0)
    es = jnp.sum(h * evenm, axis=0, keepdims=True)
    os_ = jnp.sum(h * oddm, axis=0, keepdims=True)
    acc_ref[...] = acc_ref[...] + jnp.concatenate([es, os_], axis=0)

    @pl.when(i == nb - 1)
    def _fin():
        pooled = acc_ref[...] / (K2 / 2)
        flat = jnp.concatenate([pooled[0:1], pooled[1:2]], axis=1)
        hid = _gelu(jnp.dot(flat, wo1_ref[...], preferred_element_type=_f32, precision=lax.Precision.HIGHEST)
                    + bo1_ref[...])
        o_ref[...] = (jnp.dot(hid, wo2_ref[...], preferred_element_type=_f32, precision=lax.Precision.HIGHEST)
                      + bo2_ref[...])


def _final_call(agga, aggb, tab, wr, wo, b, wo1, bo1, wo2, bo2):
    return pl.pallas_call(
        _final_body,
        grid=(P2 // 128,),
        in_specs=[
            pl.BlockSpec((2, 128, 128), lambda i: (0, i, 0)),
            pl.BlockSpec((2, 128, 128), lambda i: (0, i, 0)),
            pl.BlockSpec((128, 256), lambda i: (i, 0)),
            pl.BlockSpec((256, 512), lambda i: (0, 0)),
            pl.BlockSpec((256, 512), lambda i: (0, 0)),
            pl.BlockSpec((1, 512), lambda i: (0, 0)),
            pl.BlockSpec((1024, 256), lambda i: (0, 0)),
            pl.BlockSpec((1, 256), lambda i: (0, 0)),
            pl.BlockSpec((256, 1), lambda i: (0, 0)),
            pl.BlockSpec((1, 1), lambda i: (0, 0)),
        ],
        out_specs=pl.BlockSpec((1, 1), lambda i: (0, 0)),
        out_shape=jax.ShapeDtypeStruct((1, 1), _f32),
        scratch_shapes=[pltpu.VMEM((2, 512), _f32)],
    )(agga, aggb, tab, wr, wo, b, wo1, bo1, wo2, bo2)


_make_sc_agg = functools.lru_cache(maxsize=None)(_make_sc_agg)
_make_sc_pool = functools.lru_cache(maxsize=None)(_make_sc_pool)
_conv_i = _make_conv(P0, 16, 128, N0, pre_projected=True)
_conv_1 = _make_conv(P1, 128, 256, K1)
_topk1 = _make_topk(128, K1)
_topk2 = _make_topk(64, K2)
_scale1 = _make_scale(P1, 128)
_scale2 = _make_scale(P2, 256)


@jax.jit
def kernel(x, edge_index, bd_rel, Wd_root, Wi_rel, bi_rel, Wi_root, p1_w,
           W1_rel, b1_rel, W1_root, p2_w, W2_rel, b2_rel, W2_root,
           Wo1, bo1, Wo2, bo2):
    src = jnp.concatenate([edge_index[0],
                           jnp.full((EPAD - E,), N0, _i32)]).reshape(EROWS, 128)
    dst = jnp.concatenate([edge_index[1],
                           jnp.full((EPAD - E,), N0, _i32)]).reshape(EROWS, 128)
    x_pad = jnp.pad(x, ((0, P0 - N0), (0, 0)))
    wd = jnp.pad(Wd_root, ((0, 0), (0, 12)))
    bd = jnp.pad(bd_rel, (0, 12)).reshape(1, 16)
    wi_rel = jnp.pad(Wi_rel, ((0, 12), (0, 0)))
    wi_root = jnp.pad(Wi_root, ((0, 12), (0, 0)))
    bi = bi_rel.reshape(1, 128)
    p1 = p1_w.reshape(1, 128)
    b1 = b1_rel.reshape(1, 256)
    p2 = p2_w.reshape(1, 256)
    b2 = b2_rel.reshape(1, 512)
    bo1r = bo1.reshape(1, 256)
    bo2r = bo2.reshape(1, 1)
    z0 = jnp.zeros((P0 // 16, 128), _f32)
    z1 = jnp.zeros((P1 // 16, 128), _f32)
    z2 = jnp.zeros((P2 // 16, 128), _f32)

    t0, g0 = _h0_call(x_pad, wd, bd, wi_rel)
    a0 = _make_sc_agg(P0, 128)(g0, src, dst, z0)
    h1, s1 = _conv_i(a0, t0, wi_rel, wi_root, bi, p1)

    s1p = jnp.concatenate([s1.reshape(-1),
                           jnp.full((16384 - P0,), -2.0, _f32)]).reshape(128, 128)
    ts1, pi1 = _topk1(s1p)
    perm1 = pi1.reshape(-1)[:KP].reshape(KP // 128, 128)
    tsc1 = ts1.reshape(-1)[:P1].reshape(P1, 1)
    hg1, src1, dst1 = _make_sc_pool(P0, K1, P0, 128)(h1, perm1, src, dst)
    t1 = _scale1(hg1[:P1], tsc1)

    a1 = _make_sc_agg(P1, 128)(t1, src1, dst1, z1)
    h2, s2 = _conv_1(a1, t1, W1_rel, W1_root, b1, p2)

    s2p = jnp.concatenate([s2.reshape(-1),
                           jnp.full((8192 - P1,), -2.0, _f32)]).reshape(64, 128)
    ts2, pi2 = _topk2(s2p)
    perm2 = pi2.reshape(KP // 128, 128)
    tsc2 = ts2.reshape(-1)[:P2].reshape(P2, 1)
    hg2, src2_, dst2_ = _make_sc_pool(P1, K2, P1, 256)(h2, perm2, src1, dst1)
    t2 = _scale2(hg2[:P2], tsc2)

    agg2 = _make_sc_agg(P2, 128)
    a2a = agg2(t2[:, :128], src2_, dst2_, z2)
    a2b = agg2(t2[:, 128:], src2_, dst2_, z2)
    out = _final_call(a2a, a2b, t2, W2_rel, W2_root, b2, Wo1, bo1r, Wo2, bo2r)
    return out.reshape(1)


# speedup vs baseline: 1.0043x; 1.0043x over previous
import functools
import math

import jax
import jax.numpy as jnp
from jax import lax
from jax.experimental import pallas as pl
from jax.experimental.pallas import tpu as pltpu
from jax.experimental.pallas import tpu_sc as plsc

N0 = 10000
E = 320000
EPAD = 327680
ECH = 80
EROWS = EPAD // 128
NW = 32

P0 = 10112
K1 = 8000
P1 = 8064
K2 = 5600
P2 = 5632
KP = 8192

_f32 = jnp.float32
_i32 = jnp.int32


def _gelu(v):
    return 0.5 * v * (1.0 + lax.erf(v * (1.0 / math.sqrt(2.0))))


def _h0_body(x_ref, w_ref, b_ref, wrel_ref, h_ref, g_ref):
    i = pl.program_id(0)
    h = jnp.dot(x_ref[...], w_ref[...], preferred_element_type=_f32, precision=lax.Precision.HIGHEST) + b_ref[...]
    h = _gelu(h)
    rv = i * 128 + lax.broadcasted_iota(_i32, (128, 1), 0)
    h = jnp.where(rv < N0, h, 0.0)
    h_ref[...] = h
    g_ref[...] = jnp.dot(h, wrel_ref[...], preferred_element_type=_f32, precision=lax.Precision.HIGHEST)


def _h0_call(x_pad, wd, bd, wrel):
    return pl.pallas_call(
        _h0_body,
        grid=(P0 // 128,),
        in_specs=[
            pl.BlockSpec((128, 128), lambda i: (i, 0)),
            pl.BlockSpec((128, 16), lambda i: (0, 0)),
            pl.BlockSpec((1, 16), lambda i: (0, 0)),
            pl.BlockSpec((16, 128), lambda i: (0, 0)),
        ],
        out_specs=[
            pl.BlockSpec((128, 16), lambda i: (i, 0)),
            pl.BlockSpec((128, 128), lambda i: (i, 0)),
        ],
        out_shape=[
            jax.ShapeDtypeStruct((P0, 16), _f32),
            jax.ShapeDtypeStruct((P0, 128), _f32),
        ],
    )(x_pad, wd, bd, wrel)


def _make_sc_agg(P, F):
    rp = P // 16
    mesh = plsc.VectorSubcoreMesh(core_axis_name="c", subcore_axis_name="s",
                                  num_cores=2, num_subcores=16)

    def body(tab, src2, dst2, zb, out, acc_sh,
             idxs, idxd, rows0, rows1, sem0, sem1, sem2, sem3):
        c = lax.axis_index("c")
        s = lax.axis_index("s")
        w = s * 2 + c
        pltpu.sync_copy(zb, acc_sh.at[pl.ds(s * rp, rp)])
        plsc.subcore_barrier()

        rows = (rows0, rows1)
        semg = (sem0, sem1)
        sems = (sem2, sem3)

        def batch(bt, carry):
            @pl.when(bt > 0)
            def _drain_batch():
                for b in range(2):
                    pltpu.make_async_copy(rows[b], acc_sh.at[idxd.at[14 + b]],
                                          sems[b]).wait()

            pltpu.sync_copy(src2.at[pl.ds(w * ECH + bt * 16, 16)], idxs)
            pltpu.sync_copy(dst2.at[pl.ds(w * ECH + bt * 16, 16)], idxd)

            def step(j, c2):
                for b in range(2):
                    @pl.when(j > 0)
                    def _drain():
                        pltpu.make_async_copy(
                            rows[b], acc_sh.at[idxd.at[2 * j + b]],
                            sems[b]).wait()
                    pltpu.async_copy(tab.at[idxs.at[2 * j + b]], rows[b],
                                     semg[b])
                for b in range(2):
                    pltpu.make_async_copy(tab.at[idxs.at[2 * j + b]],
                                          rows[b], semg[b]).wait()
                    pltpu.async_copy(rows[b], acc_sh.at[idxd.at[2 * j + b]],
                                     sems[b], add=True)
                return c2

            lax.fori_loop(0, 8, step, 0)
            return carry

        lax.fori_loop(0, ECH // 16, batch, 0)
        for b in range(2):
            pltpu.make_async_copy(rows[b], acc_sh.at[idxd.at[14 + b]],
                                  sems[b]).wait()
        plsc.subcore_barrier()
        pltpu.sync_copy(acc_sh.at[pl.ds(s * rp, rp)],
                        out.at[c, pl.ds(s * rp, rp)])

    return pl.kernel(
        body,
        out_type=jax.ShapeDtypeStruct((2, P, F), _f32),
        mesh=mesh,
        scratch_types=[
            pltpu.VMEM_SHARED((P, F), _f32),
            pltpu.VMEM((16, 128), _i32),
            pltpu.VMEM((16, 128), _i32),
            pltpu.VMEM((128, F), _f32),
            pltpu.VMEM((128, F), _f32),
            pltpu.SemaphoreType.DMA,
            pltpu.SemaphoreType.DMA,
            pltpu.SemaphoreType.DMA,
            pltpu.SemaphoreType.DMA,
        ],
        compiler_params=pltpu.CompilerParams(needs_layout_passes=False),
    )


def _make_conv(P, Fin, Fout, nvalid, pre_projected=False):
    def body(agg_ref, t_ref, wr_ref, wo_ref, b_ref, pw_ref, h_ref, s_ref):
        i = pl.program_id(0)
        a = agg_ref[0] + agg_ref[1]
        if pre_projected:
            rel = a
        else:
            rel = jnp.dot(a, wr_ref[...], preferred_element_type=_f32, precision=lax.Precision.HIGHEST)
        h = (rel + b_ref[...]
             + jnp.dot(t_ref[...], wo_ref[...], preferred_element_type=_f32, precision=lax.Precision.HIGHEST))
        h = _gelu(h)
        h_ref[...] = h
        w = pw_ref[...]
        norm = jnp.sqrt(jnp.sum(w * w))
        sc = jnp.tanh(jnp.sum(h * w, axis=1, keepdims=True) / norm)
        rv = i * 128 + lax.broadcasted_iota(_i32, (128, 1), 0)
        s_ref[...] = jnp.where(rv < nvalid, sc, -2.0)

    Fa = Fout if pre_projected else Fin

    def call(agg, tab, wr, wo, b, pw):
        return pl.pallas_call(
            body,
            grid=(P // 128,),
            in_specs=[
                pl.BlockSpec((2, 128, Fa), lambda i: (0, i, 0)),
                pl.BlockSpec((128, Fin), lambda i: (i, 0)),
                pl.BlockSpec((Fin, Fout), lambda i: (0, 0)),
                pl.BlockSpec((Fin, Fout), lambda i: (0, 0)),
                pl.BlockSpec((1, Fout), lambda i: (0, 0)),
                pl.BlockSpec((1, Fout), lambda i: (0, 0)),
            ],
            out_specs=[
                pl.BlockSpec((128, Fout), lambda i: (i, 0)),
                pl.BlockSpec((128, 1), lambda i: (i, 0)),
            ],
            out_shape=[
                jax.ShapeDtypeStruct((P, Fout), _f32),
                jax.ShapeDtypeStruct((P, 1), _f32),
            ],
        )(agg, tab, wr, wo, b, pw)

    return call


def _make_topk(R, K):
    N = R * 128
    m = int(math.log2(N))
    assert 1 << m == N

    def body(s_ref, so_ref, io_ref):
        row = lax.broadcasted_iota(_i32, (R, 128), 0)
        col = lax.broadcasted_iota(_i32, (R, 128), 1)
        idx = row * 128 + col
        cs = s_ref[...]
        ci = idx

        for ke in range(1, m + 1):
            for je in range(ke - 1, -1, -1):
                d = 1 << je
                lower = (idx & d) == 0
                up = (idx & (1 << ke)) == 0
                take_first = up == lower
                if d < 128:
                    bs_dn = pltpu.roll(cs, 128 - d, 1)
                    bs_up = pltpu.roll(cs, d, 1)
                    bi_dn = pltpu.roll(ci, 128 - d, 1)
                    bi_up = pltpu.roll(ci, d, 1)
                else:
                    dd = d // 128
                    bs_dn = jnp.concatenate([cs[dd:], cs[:dd]], axis=0)
                    bs_up = jnp.concatenate([cs[R - dd:], cs[:R - dd]], axis=0)
                    bi_dn = jnp.concatenate([ci[dd:], ci[:dd]], axis=0)
                    bi_up = jnp.concatenate([ci[R - dd:], ci[:R - dd]], axis=0)
                bs = jnp.where(lower, bs_dn, bs_up)
                bi = jnp.where(lower, bi_dn, bi_up)
                before = (cs > bs) | ((cs == bs) & (ci < bi))
                sel_a = before == take_first
                cs = jnp.where(sel_a, cs, bs)
                ci = jnp.where(sel_a, ci, bi)

        keep = idx < K
        so_ref[...] = jnp.where(keep, cs, 0.0)
        io_ref[...] = jnp.where(keep, ci, 0)

    def call(scores2d):
        return pl.pallas_call(
            body,
            out_shape=[
                jax.ShapeDtypeStruct((R, 128), _f32),
                jax.ShapeDtypeStruct((R, 128), _i32),
            ],
        )(scores2d)

    return call


def _make_sc_pool(NI, K, Pprev, F):
    mesh = plsc.VectorSubcoreMesh(core_axis_name="c", subcore_axis_name="s",
                                  num_cores=2, num_subcores=16)
    n16 = NI // 16
    k16 = K // 16
    g16 = ECH * 8

    def body(htab, perm2, src2, dst2, hg, srco, dsto,
             nidx_v, perm_v, ebs, ebd, eos, eod, rows, sem):
        c = lax.axis_index("c")
        s = lax.axis_index("s")
        w = s * 2 + c

        pltpu.sync_copy(perm2, perm_v)

        def fill(t, carry):
            nidx_v[pl.ds(t * 16, 16)] = jnp.full((16,), K, _i32)
            return carry

        lax.fori_loop(0, n16, fill, 0)

        def scat(t, carry):
            r = t // 8
            cseg = (t % 8) * 16
            pv = perm_v[r, pl.ds(cseg, 16)]
            vals = t * 16 + lax.iota(_i32, 16)
            plsc.store_scatter(nidx_v, [pv], vals)
            return carry

        lax.fori_loop(0, k16, scat, 0)

        pltpu.sync_copy(src2.at[pl.ds(w * ECH, ECH)], ebs)
        pltpu.sync_copy(dst2.at[pl.ds(w * ECH, ECH)], ebd)

        def remap(t, carry):
            r = t // 8
            cseg = (t % 8) * 16
            sv = ebs[r, pl.ds(cseg, 16)]
            eos[r, pl.ds(cseg, 16)] = plsc.load_gather(nidx_v, [sv])
            dv = ebd[r, pl.ds(cseg, 16)]
            eod[r, pl.ds(cseg, 16)] = plsc.load_gather(nidx_v, [dv])
            return carry

        lax.fori_loop(0, g16, remap, 0)

        pltpu.sync_copy(eos, srco.at[pl.ds(w * ECH, ECH)])
        pltpu.sync_copy(eod, dsto.at[pl.ds(w * ECH, ECH)])

        for t in range(KP // 128 // NW):
            cid = w * (KP // 128 // NW) + t
            pltpu.async_copy(htab.at[perm_v.at[cid]], rows, sem).wait()
            pltpu.sync_copy(rows, hg.at[pl.ds(cid * 128, 128)])

    return pl.kernel(
        body,
        out_type=[
            jax.ShapeDtypeStruct((KP, F), _f32),
            jax.ShapeDtypeStruct((EROWS, 128), _i32),
            jax.ShapeDtypeStruct((EROWS, 128), _i32),
        ],
        mesh=mesh,
        scratch_types=[
            pltpu.VMEM((NI,), _i32),
            pltpu.VMEM((KP // 128, 128), _i32),
            pltpu.VMEM((ECH, 128), _i32),
            pltpu.VMEM((ECH, 128), _i32),
            pltpu.VMEM((ECH, 128), _i32),
            pltpu.VMEM((ECH, 128), _i32),
            pltpu.VMEM((128, F), _f32),
            pltpu.SemaphoreType.DMA,
        ],
        compiler_params=pltpu.CompilerParams(needs_layout_passes=False),
    )


def _make_scale(P, F):
    def body(hg_ref, ts_ref, o_ref):
        o_ref[...] = hg_ref[...] * ts_ref[...]

    def call(hg, ts):
        return pl.pallas_call(
            body,
            grid=(P // 128,),
            in_specs=[
                pl.BlockSpec((128, F), lambda i: (i, 0)),
                pl.BlockSpec((128, 1), lambda i: (i, 0)),
            ],
            out_specs=pl.BlockSpec((128, F), lambda i: (i, 0)),
            out_shape=jax.ShapeDtypeStruct((P, F), _f32),
        )(hg, ts)

    return call


def _final_body(agga_ref, aggb_ref, t_ref, wr_ref, wo_ref, b_ref,
                wo1_ref, bo1_ref, wo2_ref, bo2_ref, o_ref, acc_ref):
    i = pl.program_id(0)
    nb = pl.num_programs(0)
    a = jnp.concatenate([agga_ref[0] + agga_ref[1],
                         aggb_ref[0] + aggb_ref[1]], axis=1)
    h = (jnp.dot(a, wr_ref[...], preferred_element_type=_f32, precision=lax.Precision.HIGHEST)
         + b_ref[...]
         + jnp.dot(t_ref[...], wo_ref[...], preferred_element_type=_f32, precision=lax.Precision.HIGHEST))
    h = _gelu(h)

    @pl.when(i == 0)
    def _init():
        acc_ref[...] = jnp.zeros((2, 512), _f32)

    rv = i * 128 + lax.broadcasted_iota(_i32, (128, 1), 0)
    valid = rv < K2
    evenm = jnp.where(valid & (rv % 2 == 0), 1.0, 0.0)
    oddm = jnp.where(valid & (rv % 2 == 1), 1.0, 0.0)
    es = jnp.sum(h * evenm, axis=0, keepdims=True)
    os_ = jnp.sum(h * oddm, axis=0, keepdims=True)
    acc_ref[...] = acc_ref[...] + jnp.concatenate([es, os_], axis=0)

    @pl.when(i == nb - 1)
    def _fin():
        pooled = acc_ref[...] / (K2 / 2)
        flat = jnp.concatenate([pooled[0:1], pooled[1:2]], axis=1)
        hid = _gelu(jnp.dot(flat, wo1_ref[...], preferred_element_type=_f32, precision=lax.Precision.HIGHEST)
                    + bo1_ref[...])
        o_ref[...] = (jnp.dot(hid, wo2_ref[...], preferred_element_type=_f32, precision=lax.Precision.HIGHEST)
                      + bo2_ref[...])


def _final_call(agga, aggb, tab, wr, wo, b, wo1, bo1, wo2, bo2):
    return pl.pallas_call(
        _final_body,
        grid=(P2 // 128,),
        in_specs=[
            pl.BlockSpec((2, 128, 128), lambda i: (0, i, 0)),
            pl.BlockSpec((2, 128, 128), lambda i: (0, i, 0)),
            pl.BlockSpec((128, 256), lambda i: (i, 0)),
            pl.BlockSpec((256, 512), lambda i: (0, 0)),
            pl.BlockSpec((256, 512), lambda i: (0, 0)),
            pl.BlockSpec((1, 512), lambda i: (0, 0)),
            pl.BlockSpec((1024, 256), lambda i: (0, 0)),
            pl.BlockSpec((1, 256), lambda i: (0, 0)),
            pl.BlockSpec((256, 1), lambda i: (0, 0)),
            pl.BlockSpec((1, 1), lambda i: (0, 0)),
        ],
        out_specs=pl.BlockSpec((1, 1), lambda i: (0, 0)),
        out_shape=jax.ShapeDtypeStruct((1, 1), _f32),
        scratch_shapes=[pltpu.VMEM((2, 512), _f32)],
    )(agga, aggb, tab, wr, wo, b, wo1, bo1, wo2, bo2)


_make_sc_agg = functools.lru_cache(maxsize=None)(_make_sc_agg)
_make_sc_pool = functools.lru_cache(maxsize=None)(_make_sc_pool)
_conv_i = _make_conv(P0, 16, 128, N0, pre_projected=True)
_conv_1 = _make_conv(P1, 128, 256, K1)
_topk1 = _make_topk(128, K1)
_topk2 = _make_topk(64, K2)
_scale1 = _make_scale(P1, 128)
_scale2 = _make_scale(P2, 256)


@jax.jit
def kernel(x, edge_index, bd_rel, Wd_root, Wi_rel, bi_rel, Wi_root, p1_w,
           W1_rel, b1_rel, W1_root, p2_w, W2_rel, b2_rel, W2_root,
           Wo1, bo1, Wo2, bo2):
    src = jnp.concatenate([edge_index[0],
                           jnp.full((EPAD - E,), N0, _i32)]).reshape(EROWS, 128)
    dst = jnp.concatenate([edge_index[1],
                           jnp.full((EPAD - E,), N0, _i32)]).reshape(EROWS, 128)
    x_pad = jnp.pad(x, ((0, P0 - N0), (0, 0)))
    wd = jnp.pad(Wd_root, ((0, 0), (0, 12)))
    bd = jnp.pad(bd_rel, (0, 12)).reshape(1, 16)
    wi_rel = jnp.pad(Wi_rel, ((0, 12), (0, 0)))
    wi_root = jnp.pad(Wi_root, ((0, 12), (0, 0)))
    bi = bi_rel.reshape(1, 128)
    p1 = p1_w.reshape(1, 128)
    b1 = b1_rel.reshape(1, 256)
    p2 = p2_w.reshape(1, 256)
    b2 = b2_rel.reshape(1, 512)
    bo1r = bo1.reshape(1, 256)
    bo2r = bo2.reshape(1, 1)
    z0 = jnp.zeros((P0 // 16, 128), _f32)
    z1 = jnp.zeros((P1 // 16, 128), _f32)
    z2 = jnp.zeros((P2 // 16, 128), _f32)

    t0, g0 = _h0_call(x_pad, wd, bd, wi_rel)
    a0 = _make_sc_agg(P0, 128)(g0, src, dst, z0)
    h1, s1 = _conv_i(a0, t0, wi_rel, wi_root, bi, p1)

    s1p = jnp.concatenate([s1.reshape(-1),
                           jnp.full((16384 - P0,), -2.0, _f32)]).reshape(128, 128)
    ts1, pi1 = _topk1(s1p)
    perm1 = pi1.reshape(-1)[:KP].reshape(KP // 128, 128)
    tsc1 = ts1.reshape(-1)[:P1].reshape(P1, 1)
    hg1, src1, dst1 = _make_sc_pool(P0, K1, P0, 128)(h1, perm1, src, dst)
    t1 = _scale1(hg1[:P1], tsc1)

    a1 = _make_sc_agg(P1, 128)(t1, src1, dst1, z1)
    h2, s2 = _conv_1(a1, t1, W1_rel, W1_root, b1, p2)

    s2p = jnp.concatenate([s2.reshape(-1),
                           jnp.full((8192 - P1,), -2.0, _f32)]).reshape(64, 128)
    ts2, pi2 = _topk2(s2p)
    perm2 = pi2.reshape(KP // 128, 128)
    tsc2 = ts2.reshape(-1)[:P2].reshape(P2, 1)
    hg2, src2_, dst2_ = _make_sc_pool(P1, K2, P1, 256)(h2, perm2, src1, dst1)
    t2 = _scale2(hg2[:P2], tsc2)

    agg2 = _make_sc_agg(P2, 128)
    a2a = agg2(t2[:, :128], src2_, dst2_, z2)
    a2b = agg2(t2[:, 128:], src2_, dst2_, z2)
    out = _final_call(a2a, a2b, t2, W2_rel, W2_root, b2, Wo1, bo1r, Wo2, bo2r)
    return out.reshape(1)


# speedup vs baseline: 11.7950x; 11.7441x over previous
import functools
import math

import jax
import jax.numpy as jnp
from jax import lax
from jax.experimental import pallas as pl
from jax.experimental.pallas import tpu as pltpu
from jax.experimental.pallas import tpu_sc as plsc

N0 = 10000
E = 320000
EPAD = 327680
ECH = 80
EROWS = EPAD // 128
NW = 32

P0 = 10112
K1 = 8000
P1 = 8064
K2 = 5600
P2 = 5632
KP = 8192

_f32 = jnp.float32
_i32 = jnp.int32


def _gelu(v):
    return 0.5 * v * (1.0 + lax.erf(v * (1.0 / math.sqrt(2.0))))


def _h0_body(x_ref, w_ref, b_ref, wrel_ref, h_ref, g_ref):
    i = pl.program_id(0)
    h = jnp.dot(x_ref[...], w_ref[...], preferred_element_type=_f32, precision=lax.Precision.HIGHEST) + b_ref[...]
    h = _gelu(h)
    rv = i * 128 + lax.broadcasted_iota(_i32, (128, 1), 0)
    h = jnp.where(rv < N0, h, 0.0)
    h_ref[...] = h
    g_ref[...] = jnp.dot(h, wrel_ref[...], preferred_element_type=_f32, precision=lax.Precision.HIGHEST)


def _h0_call(x_pad, wd, bd, wrel):
    return pl.pallas_call(
        _h0_body,
        grid=(P0 // 128,),
        in_specs=[
            pl.BlockSpec((128, 128), lambda i: (i, 0)),
            pl.BlockSpec((128, 16), lambda i: (0, 0)),
            pl.BlockSpec((1, 16), lambda i: (0, 0)),
            pl.BlockSpec((16, 128), lambda i: (0, 0)),
        ],
        out_specs=[
            pl.BlockSpec((128, 16), lambda i: (i, 0)),
            pl.BlockSpec((128, 128), lambda i: (i, 0)),
        ],
        out_shape=[
            jax.ShapeDtypeStruct((P0, 16), _f32),
            jax.ShapeDtypeStruct((P0, 128), _f32),
        ],
    )(x_pad, wd, bd, wrel)


def _make_sc_agg(P, F):
    rp = P // 16
    mesh = plsc.VectorSubcoreMesh(core_axis_name="c", subcore_axis_name="s",
                                  num_cores=2, num_subcores=16)

    def body(tab, src2, dst2, zb, out, acc_sh,
             idxs, idxd, rows0, rows1, sem0, sem1, sem2, sem3):
        c = lax.axis_index("c")
        s = lax.axis_index("s")
        w = s * 2 + c
        pltpu.sync_copy(zb, acc_sh.at[pl.ds(s * rp, rp)])
        plsc.subcore_barrier()

        rows = (rows0, rows1)
        semg = (sem0, sem1)
        sems = (sem2, sem3)

        def batch(bt, carry):
            @pl.when(bt > 0)
            def _drain_batch():
                for b in range(2):
                    pltpu.make_async_copy(rows[b], acc_sh.at[idxd.at[14 + b]],
                                          sems[b]).wait()

            pltpu.sync_copy(src2.at[pl.ds(w * ECH + bt * 16, 16)], idxs)
            pltpu.sync_copy(dst2.at[pl.ds(w * ECH + bt * 16, 16)], idxd)

            def step(j, c2):
                for b in range(2):
                    @pl.when(j > 0)
                    def _drain():
                        pltpu.make_async_copy(
                            rows[b], acc_sh.at[idxd.at[2 * j + b]],
                            sems[b]).wait()
                    pltpu.async_copy(tab.at[idxs.at[2 * j + b]], rows[b],
                                     semg[b])
                for b in range(2):
                    pltpu.make_async_copy(tab.at[idxs.at[2 * j + b]],
                                          rows[b], semg[b]).wait()
                    pltpu.async_copy(rows[b], acc_sh.at[idxd.at[2 * j + b]],
                                     sems[b], add=True)
                return c2

            lax.fori_loop(0, 8, step, 0)
            return carry

        lax.fori_loop(0, ECH // 16, batch, 0)
        for b in range(2):
            pltpu.make_async_copy(rows[b], acc_sh.at[idxd.at[14 + b]],
                                  sems[b]).wait()
        plsc.subcore_barrier()
        pltpu.sync_copy(acc_sh.at[pl.ds(s * rp, rp)],
                        out.at[c, pl.ds(s * rp, rp)])

    return pl.kernel(
        body,
        out_type=jax.ShapeDtypeStruct((2, P, F), _f32),
        mesh=mesh,
        scratch_types=[
            pltpu.VMEM_SHARED((P, F), _f32),
            pltpu.VMEM((16, 128), _i32),
            pltpu.VMEM((16, 128), _i32),
            pltpu.VMEM((128, F), _f32),
            pltpu.VMEM((128, F), _f32),
            pltpu.SemaphoreType.DMA,
            pltpu.SemaphoreType.DMA,
            pltpu.SemaphoreType.DMA,
            pltpu.SemaphoreType.DMA,
        ],
        compiler_params=pltpu.CompilerParams(needs_layout_passes=False),
    )


def _make_conv(P, Fin, Fout, nvalid, pre_projected=False):
    def body(agg_ref, t_ref, wr_ref, wo_ref, b_ref, pw_ref, h_ref, s_ref):
        i = pl.program_id(0)
        a = agg_ref[0] + agg_ref[1]
        if pre_projected:
            rel = a
        else:
            rel = jnp.dot(a, wr_ref[...], preferred_element_type=_f32, precision=lax.Precision.HIGHEST)
        h = (rel + b_ref[...]
             + jnp.dot(t_ref[...], wo_ref[...], preferred_element_type=_f32, precision=lax.Precision.HIGHEST))
        h = _gelu(h)
        h_ref[...] = h
        w = pw_ref[...]
        norm = jnp.sqrt(jnp.sum(w * w))
        sc = jnp.tanh(jnp.sum(h * w, axis=1, keepdims=True) / norm)
        rv = i * 128 + lax.broadcasted_iota(_i32, (128, 1), 0)
        s_ref[...] = jnp.where(rv < nvalid, sc, -2.0)

    Fa = Fout if pre_projected else Fin

    def call(agg, tab, wr, wo, b, pw):
        return pl.pallas_call(
            body,
            grid=(P // 128,),
            in_specs=[
                pl.BlockSpec((2, 128, Fa), lambda i: (0, i, 0)),
                pl.BlockSpec((128, Fin), lambda i: (i, 0)),
                pl.BlockSpec((Fin, Fout), lambda i: (0, 0)),
                pl.BlockSpec((Fin, Fout), lambda i: (0, 0)),
                pl.BlockSpec((1, Fout), lambda i: (0, 0)),
                pl.BlockSpec((1, Fout), lambda i: (0, 0)),
            ],
            out_specs=[
                pl.BlockSpec((128, Fout), lambda i: (i, 0)),
                pl.BlockSpec((128, 1), lambda i: (i, 0)),
            ],
            out_shape=[
                jax.ShapeDtypeStruct((P, Fout), _f32),
                jax.ShapeDtypeStruct((P, 1), _f32),
            ],
        )(agg, tab, wr, wo, b, pw)

    return call


def _make_topk(R, K):
    N = R * 128
    m = int(math.log2(N))
    assert 1 << m == N

    def body(s_ref, so_ref, io_ref):
        row = lax.broadcasted_iota(_i32, (R, 128), 0)
        col = lax.broadcasted_iota(_i32, (R, 128), 1)
        idx = row * 128 + col
        cs = s_ref[...]
        ci = idx

        for ke in range(1, m + 1):
            for je in range(ke - 1, -1, -1):
                d = 1 << je
                lower = (idx & d) == 0
                up = (idx & (1 << ke)) == 0
                take_first = up == lower
                if d < 128:
                    bs_dn = pltpu.roll(cs, 128 - d, 1)
                    bs_up = pltpu.roll(cs, d, 1)
                    bi_dn = pltpu.roll(ci, 128 - d, 1)
                    bi_up = pltpu.roll(ci, d, 1)
                else:
                    dd = d // 128
                    bs_dn = jnp.concatenate([cs[dd:], cs[:dd]], axis=0)
                    bs_up = jnp.concatenate([cs[R - dd:], cs[:R - dd]], axis=0)
                    bi_dn = jnp.concatenate([ci[dd:], ci[:dd]], axis=0)
                    bi_up = jnp.concatenate([ci[R - dd:], ci[:R - dd]], axis=0)
                bs = jnp.where(lower, bs_dn, bs_up)
                bi = jnp.where(lower, bi_dn, bi_up)
                before = (cs > bs) | ((cs == bs) & (ci < bi))
                sel_a = before == take_first
                cs = jnp.where(sel_a, cs, bs)
                ci = jnp.where(sel_a, ci, bi)

        keep = idx < K
        so_ref[...] = jnp.where(keep, cs, 0.0)
        io_ref[...] = jnp.where(keep, ci, 0)

    def call(scores2d):
        return pl.pallas_call(
            body,
            out_shape=[
                jax.ShapeDtypeStruct((R, 128), _f32),
                jax.ShapeDtypeStruct((R, 128), _i32),
            ],
        )(scores2d)

    return call


def _make_sc_pool(NI, K, Pprev, F, DN):
    mesh = plsc.VectorSubcoreMesh(core_axis_name="c", subcore_axis_name="s",
                                  num_cores=2, num_subcores=16)
    n16 = NI // 16
    k16 = K // 16
    g16 = ECH * 8

    def body(htab, perm2, src2, dst2, hg, srco, dsto,
             nidx_v, perm_v, ebs, ebd, eos, eod, rows, sem):
        c = lax.axis_index("c")
        s = lax.axis_index("s")
        w = s * 2 + c

        pltpu.sync_copy(perm2, perm_v)

        def fill(t, carry):
            nidx_v[pl.ds(t * 16, 16)] = jnp.full((16,), K, _i32)
            return carry

        lax.fori_loop(0, n16, fill, 0)

        def scat(t, carry):
            r = t // 8
            cseg = (t % 8) * 16
            pv = perm_v[r, pl.ds(cseg, 16)]
            vals = t * 16 + lax.iota(_i32, 16)
            plsc.store_scatter(nidx_v, [pv], vals)
            return carry

        lax.fori_loop(0, k16, scat, 0)

        pltpu.sync_copy(src2.at[pl.ds(w * ECH, ECH)], ebs)
        pltpu.sync_copy(dst2.at[pl.ds(w * ECH, ECH)], ebd)

        def remap(t, carry):
            r = t // 8
            cseg = (t % 8) * 16
            spread = K + ((t * 16 + lax.iota(_i32, 16)) & (DN - 1))
            sv = ebs[r, pl.ds(cseg, 16)]
            g = plsc.load_gather(nidx_v, [sv])
            eos[r, pl.ds(cseg, 16)] = jnp.where(g == K, spread, g)
            dv = ebd[r, pl.ds(cseg, 16)]
            g2 = plsc.load_gather(nidx_v, [dv])
            eod[r, pl.ds(cseg, 16)] = jnp.where(g2 == K, spread, g2)
            return carry

        lax.fori_loop(0, g16, remap, 0)

        pltpu.sync_copy(eos, srco.at[pl.ds(w * ECH, ECH)])
        pltpu.sync_copy(eod, dsto.at[pl.ds(w * ECH, ECH)])

        for t in range(KP // 128 // NW):
            cid = w * (KP // 128 // NW) + t
            pltpu.async_copy(htab.at[perm_v.at[cid]], rows, sem).wait()
            pltpu.sync_copy(rows, hg.at[pl.ds(cid * 128, 128)])

    return pl.kernel(
        body,
        out_type=[
            jax.ShapeDtypeStruct((KP, F), _f32),
            jax.ShapeDtypeStruct((EROWS, 128), _i32),
            jax.ShapeDtypeStruct((EROWS, 128), _i32),
        ],
        mesh=mesh,
        scratch_types=[
            pltpu.VMEM((NI,), _i32),
            pltpu.VMEM((KP // 128, 128), _i32),
            pltpu.VMEM((ECH, 128), _i32),
            pltpu.VMEM((ECH, 128), _i32),
            pltpu.VMEM((ECH, 128), _i32),
            pltpu.VMEM((ECH, 128), _i32),
            pltpu.VMEM((128, F), _f32),
            pltpu.SemaphoreType.DMA,
        ],
        compiler_params=pltpu.CompilerParams(needs_layout_passes=False),
    )


def _make_scale(P, F):
    def body(hg_ref, ts_ref, o_ref):
        o_ref[...] = hg_ref[...] * ts_ref[...]

    def call(hg, ts):
        return pl.pallas_call(
            body,
            grid=(P // 128,),
            in_specs=[
                pl.BlockSpec((128, F), lambda i: (i, 0)),
                pl.BlockSpec((128, 1), lambda i: (i, 0)),
            ],
            out_specs=pl.BlockSpec((128, F), lambda i: (i, 0)),
            out_shape=jax.ShapeDtypeStruct((P, F), _f32),
        )(hg, ts)

    return call


def _final_body(agga_ref, aggb_ref, t_ref, wr_ref, wo_ref, b_ref,
                wo1_ref, bo1_ref, wo2_ref, bo2_ref, o_ref, acc_ref):
    i = pl.program_id(0)
    nb = pl.num_programs(0)
    a = jnp.concatenate([agga_ref[0] + agga_ref[1],
                         aggb_ref[0] + aggb_ref[1]], axis=1)
    h = (jnp.dot(a, wr_ref[...], preferred_element_type=_f32, precision=lax.Precision.HIGHEST)
         + b_ref[...]
         + jnp.dot(t_ref[...], wo_ref[...], preferred_element_type=_f32, precision=lax.Precision.HIGHEST))
    h = _gelu(h)

    @pl.when(i == 0)
    def _init():
        acc_ref[...] = jnp.zeros((2, 512), _f32)

    rv = i * 128 + lax.broadcasted_iota(_i32, (128, 1), 0)
    valid = rv < K2
    evenm = jnp.where(valid & (rv % 2 == 0), 1.0, 0.0)
    oddm = jnp.where(valid & (rv % 2 == 1), 1.0, 0.0)
    es = jnp.sum(h * evenm, axis=0, keepdims=True)
    os_ = jnp.sum(h * oddm, axis=0, keepdims=True)
    acc_ref[...] = acc_ref[...] + jnp.concatenate([es, os_], axis=0)

    @pl.when(i == nb - 1)
    def _fin():
        pooled = acc_ref[...] / (K2 / 2)
        flat = jnp.concatenate([pooled[0:1], pooled[1:2]], axis=1)
        hid = _gelu(jnp.dot(flat, wo1_ref[...], preferred_element_type=_f32, precision=lax.Precision.HIGHEST)
                    + bo1_ref[...])
        o_ref[...] = (jnp.dot(hid, wo2_ref[...], preferred_element_type=_f32, precision=lax.Precision.HIGHEST)
                      + bo2_ref[...])


def _final_call(agga, aggb, tab, wr, wo, b, wo1, bo1, wo2, bo2):
    return pl.pallas_call(
        _final_body,
        grid=(P2 // 128,),
        in_specs=[
            pl.BlockSpec((2, 128, 128), lambda i: (0, i, 0)),
            pl.BlockSpec((2, 128, 128), lambda i: (0, i, 0)),
            pl.BlockSpec((128, 256), lambda i: (i, 0)),
            pl.BlockSpec((256, 512), lambda i: (0, 0)),
            pl.BlockSpec((256, 512), lambda i: (0, 0)),
            pl.BlockSpec((1, 512), lambda i: (0, 0)),
            pl.BlockSpec((1024, 256), lambda i: (0, 0)),
            pl.BlockSpec((1, 256), lambda i: (0, 0)),
            pl.BlockSpec((256, 1), lambda i: (0, 0)),
            pl.BlockSpec((1, 1), lambda i: (0, 0)),
        ],
        out_specs=pl.BlockSpec((1, 1), lambda i: (0, 0)),
        out_shape=jax.ShapeDtypeStruct((1, 1), _f32),
        scratch_shapes=[pltpu.VMEM((2, 512), _f32)],
    )(agga, aggb, tab, wr, wo, b, wo1, bo1, wo2, bo2)


_make_sc_agg = functools.lru_cache(maxsize=None)(_make_sc_agg)
_make_sc_pool = functools.lru_cache(maxsize=None)(_make_sc_pool)
_conv_i = _make_conv(P0, 16, 128, N0, pre_projected=True)
_conv_1 = _make_conv(P1, 128, 256, K1)
_topk1 = _make_topk(128, K1)
_topk2 = _make_topk(64, K2)
_scale1 = _make_scale(P1, 128)
_scale2 = _make_scale(P2, 256)


@jax.jit
def kernel(x, edge_index, bd_rel, Wd_root, Wi_rel, bi_rel, Wi_root, p1_w,
           W1_rel, b1_rel, W1_root, p2_w, W2_rel, b2_rel, W2_root,
           Wo1, bo1, Wo2, bo2):
    padidx = N0 + (jnp.arange(EPAD - E, dtype=_i32) & 63)
    src = jnp.concatenate([edge_index[0], padidx]).reshape(EROWS, 128)
    dst = jnp.concatenate([edge_index[1], padidx]).reshape(EROWS, 128)
    x_pad = jnp.pad(x, ((0, P0 - N0), (0, 0)))
    wd = jnp.pad(Wd_root, ((0, 0), (0, 12)))
    bd = jnp.pad(bd_rel, (0, 12)).reshape(1, 16)
    wi_rel = jnp.pad(Wi_rel, ((0, 12), (0, 0)))
    wi_root = jnp.pad(Wi_root, ((0, 12), (0, 0)))
    bi = bi_rel.reshape(1, 128)
    p1 = p1_w.reshape(1, 128)
    b1 = b1_rel.reshape(1, 256)
    p2 = p2_w.reshape(1, 256)
    b2 = b2_rel.reshape(1, 512)
    bo1r = bo1.reshape(1, 256)
    bo2r = bo2.reshape(1, 1)
    z0 = jnp.zeros((P0 // 16, 128), _f32)
    z1 = jnp.zeros((P1 // 16, 128), _f32)
    z2 = jnp.zeros((P2 // 16, 128), _f32)

    t0, g0 = _h0_call(x_pad, wd, bd, wi_rel)
    a0 = _make_sc_agg(P0, 128)(g0, src, dst, z0)
    h1, s1 = _conv_i(a0, t0, wi_rel, wi_root, bi, p1)

    s1p = jnp.concatenate([s1.reshape(-1),
                           jnp.full((16384 - P0,), -2.0, _f32)]).reshape(128, 128)
    ts1, pi1 = _topk1(s1p)
    perm1 = pi1.reshape(-1)[:KP].reshape(KP // 128, 128)
    tsc1 = ts1.reshape(-1)[:P1].reshape(P1, 1)
    hg1, src1, dst1 = _make_sc_pool(P0, K1, P0, 128, 64)(h1, perm1, src, dst)
    t1 = _scale1(hg1[:P1], tsc1)

    a1 = _make_sc_agg(P1, 128)(t1, src1, dst1, z1)
    h2, s2 = _conv_1(a1, t1, W1_rel, W1_root, b1, p2)

    s2p = jnp.concatenate([s2.reshape(-1),
                           jnp.full((8192 - P1,), -2.0, _f32)]).reshape(64, 128)
    ts2, pi2 = _topk2(s2p)
    perm2 = pi2.reshape(KP // 128, 128)
    tsc2 = ts2.reshape(-1)[:P2].reshape(P2, 1)
    hg2, src2_, dst2_ = _make_sc_pool(P1, K2, P1, 256, 32)(h2, perm2, src1,
                                                           dst1)
    t2 = _scale2(hg2[:P2], tsc2)

    agg2 = _make_sc_agg(P2, 128)
    a2a = agg2(t2[:, :128], src2_, dst2_, z2)
    a2b = agg2(t2[:, 128:], src2_, dst2_, z2)
    out = _final_call(a2a, a2b, t2, W2_rel, W2_root, b2, Wo1, bo1r, Wo2, bo2r)
    return out.reshape(1)
